# Initial kernel scaffold; baseline (speedup 1.0000x reference)
import jax
import jax.numpy as jnp
from jax.experimental import pallas as pl


def kernel(inputs, w):
    raise NotImplementedError("write your pallas kernel here")


# speedup vs baseline: 1.3267x; 1.3267x over previous
import jax
import jax.numpy as jnp
from jax import lax
from jax.experimental import pallas as pl
from jax.experimental.pallas import tpu as pltpu
from jax.experimental.pallas import tpu_sc as plsc

NUM_CORES = 2
NUM_SUBCORES = 16
NUM_WORKERS = NUM_CORES * NUM_SUBCORES
LANES = 16


def _make_sc_kernel(batch, n_fields):
    rows_per_w = batch // NUM_WORKERS
    idx_per_w = rows_per_w * n_fields
    blocks_per_w = rows_per_w // LANES

    def body(idx_hbm, w_hbm, out_hbm, idx_v, vals_v, out_v, sem):
        wid = lax.axis_index("s") * NUM_CORES + lax.axis_index("c")
        ibase = wid * idx_per_w
        obase = wid * rows_per_w
        pltpu.sync_copy(idx_hbm.at[pl.ds(ibase, idx_per_w)], idx_v)
        pltpu.async_copy(w_hbm.at[idx_v], vals_v, sem).wait()
        lane_off = lax.iota(jnp.int32, LANES) * n_fields

        def block(i, carry):
            base = i * LANES
            pos0 = base * n_fields + lane_off
            acc = plsc.load_gather(vals_v, [pos0])
            for f in range(1, n_fields):
                acc = acc + plsc.load_gather(vals_v, [pos0 + f])
            out_v[pl.ds(base, LANES)] = acc
            return carry

        lax.fori_loop(0, blocks_per_w, block, 0)
        pltpu.sync_copy(out_v, out_hbm.at[pl.ds(obase, rows_per_w)])

    mesh = plsc.VectorSubcoreMesh(core_axis_name="c", subcore_axis_name="s")
    return pl.kernel(
        body,
        out_type=jax.ShapeDtypeStruct((batch,), jnp.float32),
        mesh=mesh,
        scratch_types=[
            pltpu.VMEM((idx_per_w,), jnp.int32),
            pltpu.VMEM((idx_per_w,), jnp.float32),
            pltpu.VMEM((rows_per_w,), jnp.float32),
            pltpu.SemaphoreType.DMA,
        ],
        compiler_params=pltpu.CompilerParams(needs_layout_passes=False),
    )


def kernel(inputs, w):
    batch, n_fields = inputs.shape
    idx = inputs.reshape(-1).astype(jnp.int32)
    table = w.reshape(-1)
    out = _make_sc_kernel(batch, n_fields)(idx, table)
    return out.reshape(batch, 1)


# speedup vs baseline: 1.3560x; 1.0221x over previous
import jax
import jax.numpy as jnp
from jax import lax
from jax.experimental import pallas as pl
from jax.experimental.pallas import tpu as pltpu
from jax.experimental.pallas import tpu_sc as plsc

NUM_CORES = 2
NUM_SUBCORES = 16
NUM_WORKERS = NUM_CORES * NUM_SUBCORES
LANES = 16


def _make_sc_kernel(batch, n_fields, vocab):
    rows_per_w = batch // NUM_WORKERS
    idx_per_w = rows_per_w * n_fields
    blocks_per_w = rows_per_w // LANES
    n_sub = 8
    sub = (vocab // (NUM_SUBCORES * n_sub)) & ~7
    chunk = sub * n_sub
    tail_off = chunk * NUM_SUBCORES
    tail = vocab - tail_off
    assert tail <= sub

    def body(idx_hbm, w_hbm, out_hbm, table_s, stage_v, idx_v, vals_v, out_v,
             sem):
        c = lax.axis_index("c")
        s = lax.axis_index("s")
        wid = s * NUM_CORES + c
        ibase = wid * idx_per_w
        obase = wid * rows_per_w

        idx_cp = pltpu.async_copy(idx_hbm.at[pl.ds(ibase, idx_per_w)], idx_v, sem)

        off = s * chunk
        for j in range(n_sub):
            pltpu.sync_copy(w_hbm.at[pl.ds(off + j * sub, sub)], stage_v)
            pltpu.sync_copy(stage_v, table_s.at[pl.ds(off + j * sub, sub)])
        if tail:
            @pl.when(s == NUM_SUBCORES - 1)
            def _():
                pltpu.sync_copy(w_hbm.at[pl.ds(tail_off, tail)],
                                stage_v.at[pl.ds(0, tail)])
                pltpu.sync_copy(stage_v.at[pl.ds(0, tail)],
                                table_s.at[pl.ds(tail_off, tail)])
        plsc.subcore_barrier()

        idx_cp.wait()
        pltpu.async_copy(table_s.at[idx_v], vals_v, sem).wait()

        lane_off = lax.iota(jnp.int32, LANES) * n_fields

        def block(i, carry):
            base = i * LANES
            pos0 = base * n_fields + lane_off
            acc = plsc.load_gather(vals_v, [pos0])
            for f in range(1, n_fields):
                acc = acc + plsc.load_gather(vals_v, [pos0 + f])
            out_v[pl.ds(base, LANES)] = acc
            return carry

        lax.fori_loop(0, blocks_per_w, block, 0)
        pltpu.sync_copy(out_v, out_hbm.at[pl.ds(obase, rows_per_w)])

    mesh = plsc.VectorSubcoreMesh(core_axis_name="c", subcore_axis_name="s")
    return pl.kernel(
        body,
        out_type=jax.ShapeDtypeStruct((batch,), jnp.float32),
        mesh=mesh,
        scratch_types=[
            pltpu.VMEM_SHARED((vocab,), jnp.float32),
            pltpu.VMEM((sub,), jnp.float32),
            pltpu.VMEM((idx_per_w,), jnp.int32),
            pltpu.VMEM((idx_per_w,), jnp.float32),
            pltpu.VMEM((rows_per_w,), jnp.float32),
            pltpu.SemaphoreType.DMA,
        ],
        compiler_params=pltpu.CompilerParams(needs_layout_passes=False),
    )


def kernel(inputs, w):
    batch, n_fields = inputs.shape
    vocab = w.shape[0]
    idx = inputs.reshape(-1).astype(jnp.int32)
    table = w.reshape(-1)
    out = _make_sc_kernel(batch, n_fields, vocab)(idx, table)
    return out.reshape(batch, 1)


# speedup vs baseline: 3.4732x; 2.5614x over previous
import jax
import jax.numpy as jnp
from jax import lax
from jax.experimental import pallas as pl
from jax.experimental.pallas import tpu as pltpu
from jax.experimental.pallas import tpu_sc as plsc

NUM_CORES = 2
NUM_SUBCORES = 16
NUM_WORKERS = NUM_CORES * NUM_SUBCORES
LANES = 16


def _make_sc_kernel(batch, n_fields, vocab):
    rows_per_w = batch // NUM_WORKERS
    idx_per_w = rows_per_w * n_fields
    blocks_per_w = rows_per_w // LANES
    n_sub = 8
    sub = (vocab // (NUM_SUBCORES * n_sub)) & ~7
    chunk = sub * n_sub
    tail_off = chunk * NUM_SUBCORES
    tail = vocab - tail_off
    assert tail <= sub

    def body(idx_hbm, w_hbm, out_hbm, table_s, stage_v, idx_v, vals_v, out_v,
             sem):
        c = lax.axis_index("c")
        s = lax.axis_index("s")
        wid = s * NUM_CORES + c
        r0 = wid * rows_per_w

        idx_copies = [
            pltpu.async_copy(idx_hbm.at[f, pl.ds(r0, rows_per_w)],
                             idx_v.at[pl.ds(f * rows_per_w, rows_per_w)], sem)
            for f in range(n_fields)
        ]

        off = s * chunk
        for j in range(n_sub):
            pltpu.sync_copy(w_hbm.at[0, pl.ds(off + j * sub, sub)], stage_v)
            pltpu.sync_copy(stage_v, table_s.at[pl.ds(off + j * sub, sub)])
        if tail:
            @pl.when(s == NUM_SUBCORES - 1)
            def _():
                pltpu.sync_copy(w_hbm.at[0, pl.ds(tail_off, tail)],
                                stage_v.at[pl.ds(0, tail)])
                pltpu.sync_copy(stage_v.at[pl.ds(0, tail)],
                                table_s.at[pl.ds(tail_off, tail)])
        plsc.subcore_barrier()

        for cp in idx_copies:
            cp.wait()
        pltpu.async_copy(table_s.at[idx_v], vals_v, sem).wait()

        def block(i, carry):
            base = i * LANES
            acc = vals_v[pl.ds(base, LANES)]
            for f in range(1, n_fields):
                acc = acc + vals_v[pl.ds(f * rows_per_w + base, LANES)]
            out_v[pl.ds(base, LANES)] = acc
            return carry

        lax.fori_loop(0, blocks_per_w, block, 0)
        pltpu.sync_copy(out_v, out_hbm.at[0, pl.ds(r0, rows_per_w)])

    mesh = plsc.VectorSubcoreMesh(core_axis_name="c", subcore_axis_name="s")
    return pl.kernel(
        body,
        out_type=jax.ShapeDtypeStruct((1, batch), jnp.float32),
        mesh=mesh,
        scratch_types=[
            pltpu.VMEM_SHARED((vocab,), jnp.float32),
            pltpu.VMEM((sub,), jnp.float32),
            pltpu.VMEM((idx_per_w,), jnp.int32),
            pltpu.VMEM((idx_per_w,), jnp.float32),
            pltpu.VMEM((rows_per_w,), jnp.float32),
            pltpu.SemaphoreType.DMA,
        ],
        compiler_params=pltpu.CompilerParams(needs_layout_passes=False),
    )


def kernel(inputs, w):
    batch, n_fields = inputs.shape
    vocab = w.shape[0]
    out = _make_sc_kernel(batch, n_fields, vocab)(inputs.T, w.T)
    return out.T


# speedup vs baseline: 3.9244x; 1.1299x over previous
import jax
import jax.numpy as jnp
from jax import lax
from jax.experimental import pallas as pl
from jax.experimental.pallas import tpu as pltpu
from jax.experimental.pallas import tpu_sc as plsc

NUM_CORES = 2
NUM_SUBCORES = 16
NUM_WORKERS = NUM_CORES * NUM_SUBCORES
LANES = 16


def _make_sc_kernel(batch, n_fields, vocab):
    rows_per_w = batch // NUM_WORKERS
    idx_per_w = rows_per_w * n_fields
    blocks_per_w = rows_per_w // LANES
    n_sub = 8
    sub = (vocab // (NUM_SUBCORES * n_sub)) & ~7
    chunk = sub * n_sub
    tail_off = chunk * NUM_SUBCORES
    tail = vocab - tail_off
    assert tail <= sub

    def body(idx_hbm, w_hbm, out_hbm, table_s, stage0, stage1, idx_v, vals_v,
             out_v, sem, sem_in0, sem_in1, sem_out0, sem_out1):
        c = lax.axis_index("c")
        s = lax.axis_index("s")
        wid = s * NUM_CORES + c
        r0 = wid * rows_per_w

        idx_copies = [
            pltpu.async_copy(idx_hbm.at[f, pl.ds(r0, rows_per_w)],
                             idx_v.at[pl.ds(f * rows_per_w, rows_per_w)], sem)
            for f in range(n_fields)
        ]

        off = s * chunk
        stage = (stage0, stage1)
        sem_in = (sem_in0, sem_in1)
        sem_out = (sem_out0, sem_out1)
        in_cp = [None, None]
        out_cp = [None, None]
        for j in range(n_sub + 1):
            b = j & 1
            if j < n_sub:
                if out_cp[b] is not None:
                    out_cp[b].wait()
                in_cp[b] = pltpu.async_copy(
                    w_hbm.at[0, pl.ds(off + j * sub, sub)],
                    stage[b], sem_in[b])
            if j >= 1:
                p = (j - 1) & 1
                in_cp[p].wait()
                out_cp[p] = pltpu.async_copy(
                    stage[p],
                    table_s.at[pl.ds(off + (j - 1) * sub, sub)], sem_out[p])
        for cp in out_cp:
            cp.wait()
        if tail:
            @pl.when(s == NUM_SUBCORES - 1)
            def _():
                pltpu.sync_copy(w_hbm.at[0, pl.ds(tail_off, tail)],
                                stage0.at[pl.ds(0, tail)])
                pltpu.sync_copy(stage0.at[pl.ds(0, tail)],
                                table_s.at[pl.ds(tail_off, tail)])
        plsc.subcore_barrier()

        for cp in idx_copies:
            cp.wait()
        pltpu.async_copy(table_s.at[idx_v], vals_v, sem).wait()

        def block(i, carry):
            base = i * LANES
            acc = vals_v[pl.ds(base, LANES)]
            for f in range(1, n_fields):
                acc = acc + vals_v[pl.ds(f * rows_per_w + base, LANES)]
            out_v[pl.ds(base, LANES)] = acc
            return carry

        lax.fori_loop(0, blocks_per_w, block, 0)
        pltpu.sync_copy(out_v, out_hbm.at[0, pl.ds(r0, rows_per_w)])

    mesh = plsc.VectorSubcoreMesh(core_axis_name="c", subcore_axis_name="s")
    return pl.kernel(
        body,
        out_type=jax.ShapeDtypeStruct((1, batch), jnp.float32),
        mesh=mesh,
        scratch_types=[
            pltpu.VMEM_SHARED((vocab,), jnp.float32),
            pltpu.VMEM((sub,), jnp.float32),
            pltpu.VMEM((sub,), jnp.float32),
            pltpu.VMEM((idx_per_w,), jnp.int32),
            pltpu.VMEM((idx_per_w,), jnp.float32),
            pltpu.VMEM((rows_per_w,), jnp.float32),
            pltpu.SemaphoreType.DMA,
            pltpu.SemaphoreType.DMA,
            pltpu.SemaphoreType.DMA,
            pltpu.SemaphoreType.DMA,
            pltpu.SemaphoreType.DMA,
        ],
        compiler_params=pltpu.CompilerParams(needs_layout_passes=False),
    )


def kernel(inputs, w):
    batch, n_fields = inputs.shape
    vocab = w.shape[0]
    out = _make_sc_kernel(batch, n_fields, vocab)(inputs.T, w.T)
    return out.T


# speedup vs baseline: 3.9722x; 1.0122x over previous
import jax
import jax.numpy as jnp
from jax import lax
from jax.experimental import pallas as pl
from jax.experimental.pallas import tpu as pltpu
from jax.experimental.pallas import tpu_sc as plsc

NUM_CORES = 2
NUM_SUBCORES = 16
NUM_WORKERS = NUM_CORES * NUM_SUBCORES
LANES = 16


def _make_sc_kernel(batch, n_fields, vocab):
    rows_per_w = batch // NUM_WORKERS
    idx_per_w = rows_per_w * n_fields
    blocks_per_w = rows_per_w // LANES
    n_sub = 8
    sub = (vocab // (NUM_SUBCORES * n_sub)) & ~7
    chunk = sub * n_sub
    tail_off = chunk * NUM_SUBCORES
    tail = vocab - tail_off
    assert tail <= sub

    def body(idx_hbm, w_hbm, out_hbm, table_s, stage0, stage1, idx_v, vals_v,
             out_v, sem, sem_in0, sem_in1, sem_out0, sem_out1, sem_g):
        c = lax.axis_index("c")
        s = lax.axis_index("s")
        wid = s * NUM_CORES + c
        r0 = wid * rows_per_w

        off = s * chunk
        stage = (stage0, stage1)
        sem_in = (sem_in0, sem_in1)
        sem_out = (sem_out0, sem_out1)
        in_cp = [None, None]
        out_cp = [None, None]
        for j in range(2):
            in_cp[j] = pltpu.async_copy(
                w_hbm.at[0, pl.ds(off + j * sub, sub)], stage[j], sem_in[j])

        idx_copies = [
            pltpu.async_copy(idx_hbm.at[f, pl.ds(r0, rows_per_w)],
                             idx_v.at[pl.ds(f * rows_per_w, rows_per_w)], sem)
            for f in range(n_fields)
        ]

        for j in range(n_sub + 1):
            b = j & 1
            if 2 <= j < n_sub:
                out_cp[b].wait()
                in_cp[b] = pltpu.async_copy(
                    w_hbm.at[0, pl.ds(off + j * sub, sub)],
                    stage[b], sem_in[b])
            if j >= 1:
                p = (j - 1) & 1
                in_cp[p].wait()
                out_cp[p] = pltpu.async_copy(
                    stage[p],
                    table_s.at[pl.ds(off + (j - 1) * sub, sub)], sem_out[p])
        for cp in out_cp:
            cp.wait()
        if tail:
            @pl.when(s == NUM_SUBCORES - 1)
            def _():
                pltpu.sync_copy(w_hbm.at[0, pl.ds(tail_off, tail)],
                                stage0.at[pl.ds(0, tail)])
                pltpu.sync_copy(stage0.at[pl.ds(0, tail)],
                                table_s.at[pl.ds(tail_off, tail)])
        plsc.subcore_barrier()

        for cp in idx_copies:
            cp.wait()
        f_half = n_fields // 2
        h1 = f_half * rows_per_w
        h2 = idx_per_w - h1
        pltpu.async_copy(table_s.at[idx_v.at[pl.ds(0, h1)]],
                         vals_v.at[pl.ds(0, h1)], sem).wait()
        g2 = pltpu.async_copy(table_s.at[idx_v.at[pl.ds(h1, h2)]],
                              vals_v.at[pl.ds(h1, h2)], sem_g)

        def block1(i, carry):
            base = i * LANES
            acc = vals_v[pl.ds(base, LANES)]
            for f in range(1, f_half):
                acc = acc + vals_v[pl.ds(f * rows_per_w + base, LANES)]
            out_v[pl.ds(base, LANES)] = acc
            return carry

        lax.fori_loop(0, blocks_per_w, block1, 0)
        g2.wait()

        def block2(i, carry):
            base = i * LANES
            acc = out_v[pl.ds(base, LANES)]
            for f in range(f_half, n_fields):
                acc = acc + vals_v[pl.ds(f * rows_per_w + base, LANES)]
            out_v[pl.ds(base, LANES)] = acc
            return carry

        lax.fori_loop(0, blocks_per_w, block2, 0)
        pltpu.sync_copy(out_v, out_hbm.at[0, pl.ds(r0, rows_per_w)])

    mesh = plsc.VectorSubcoreMesh(core_axis_name="c", subcore_axis_name="s")
    return pl.kernel(
        body,
        out_type=jax.ShapeDtypeStruct((1, batch), jnp.float32),
        mesh=mesh,
        scratch_types=[
            pltpu.VMEM_SHARED((vocab,), jnp.float32),
            pltpu.VMEM((sub,), jnp.float32),
            pltpu.VMEM((sub,), jnp.float32),
            pltpu.VMEM((idx_per_w,), jnp.int32),
            pltpu.VMEM((idx_per_w,), jnp.float32),
            pltpu.VMEM((rows_per_w,), jnp.float32),
            pltpu.SemaphoreType.DMA,
            pltpu.SemaphoreType.DMA,
            pltpu.SemaphoreType.DMA,
            pltpu.SemaphoreType.DMA,
            pltpu.SemaphoreType.DMA,
            pltpu.SemaphoreType.DMA,
        ],
        compiler_params=pltpu.CompilerParams(needs_layout_passes=False),
    )


def kernel(inputs, w):
    batch, n_fields = inputs.shape
    vocab = w.shape[0]
    out = _make_sc_kernel(batch, n_fields, vocab)(inputs.T, w.T)
    return out.T
